# baseline (speedup 1.0000x reference)
import functools

import jax
import jax.numpy as jnp
from jax import lax
from jax.experimental import pallas as pl
from jax.experimental.pallas import tpu as pltpu
from jax.experimental.pallas import tpu_sc as plsc

N = 10000
E = 320000
H = 256
HH = 128
G = 64
L = 3

NS = 16
C = 40
NCH = 500
EPC = NCH * C
EPAD = NS * EPC
NGARB = 0
ACCR = N + NGARB
ROWS = 624
ZR = 16
NSTG = 4

RB = 1000
NB = N // RB


def _make_sc_agg():
    mesh = plsc.VectorSubcoreMesh(core_axis_name="c", subcore_axis_name="s")

    @functools.partial(
        pl.kernel,
        mesh=mesh,
        out_type=jax.ShapeDtypeStruct((2 * N, HH), jnp.float32),
        scratch_types=[
            pltpu.VMEM((C,), jnp.int32),
            pltpu.VMEM((C,), jnp.int32),
            pltpu.VMEM((C,), jnp.int32),
            pltpu.VMEM((C,), jnp.int32),
            pltpu.VMEM((C,), jnp.int32),
            pltpu.VMEM((C,), jnp.int32),
            pltpu.VMEM((C,), jnp.int32),
            pltpu.VMEM((C,), jnp.int32),
            pltpu.VMEM((C, HH), jnp.float32),
            pltpu.VMEM((C, HH), jnp.float32),
            pltpu.VMEM((ZR, HH), jnp.float32),
            pltpu.VMEM_SHARED((ACCR, HH), jnp.float32),
            pltpu.SemaphoreType.DMA,
            pltpu.SemaphoreType.DMA,
            pltpu.SemaphoreType.DMA,
            pltpu.SemaphoreType.DMA,
            pltpu.SemaphoreType.DMA,
            pltpu.SemaphoreType.DMA,
            pltpu.SemaphoreType.DMA,
            pltpu.SemaphoreType.DMA,
        ],
    )
    def sc_agg(src2_hbm, dst_hbm, h_hbm, out_hbm,
               srcb0, srcb1, srcb2, srcb3, dstb0, dstb1, dstb2, dstb3,
               stage0, stage1, zbuf, acc,
               semg0, semg1, sems0, sems1, semi0, semi1, semi2, semi3):
        cid = lax.axis_index("c")
        sid = lax.axis_index("s")
        srcb = (srcb0, srcb1, srcb2, srcb3)
        dstb = (dstb0, dstb1, dstb2, dstb3)
        stage = (stage0, stage1)
        semg = (semg0, semg1)
        sems = (sems0, sems1)
        semi = (semi0, semi1, semi2, semi3)

        def zrow(i, _):
            def zcol(j, _):
                zbuf[i, pl.ds(j * 16, 16)] = jnp.zeros((16,), jnp.float32)
                return 0
            return lax.fori_loop(0, HH // 16, zcol, 0)
        lax.fori_loop(0, ZR, zrow, 0)

        def zcp(t, _):
            pltpu.make_async_copy(
                zbuf, acc.at[pl.ds(sid * ROWS + t * ZR, ZR)], sems0).start()
            return 0
        lax.fori_loop(0, ROWS // ZR, zcp, 0)

        @pl.when(sid == NS - 1)
        def _():
            pltpu.make_async_copy(
                zbuf, acc.at[pl.ds(NS * ROWS, N - NS * ROWS)], sems0).start()

        def zwait(t, _):
            pltpu.make_async_copy(
                zbuf, acc.at[pl.ds(0, ZR)], sems0).wait()
            return 0
        lax.fori_loop(0, ROWS // ZR, zwait, 0)

        @pl.when(sid == NS - 1)
        def _():
            pltpu.make_async_copy(
                zbuf, acc.at[pl.ds(0, N - NS * ROWS)], sems0).wait()

        plsc.subcore_barrier()


        def start_idx(k, q):
            kk = jnp.minimum(k, NCH - 1)
            base = sid * EPC + kk * C
            pltpu.make_async_copy(
                src2_hbm.at[pl.ds(cid * EPAD + base, C)], srcb[q], semi[q]).start()
            pltpu.make_async_copy(
                dst_hbm.at[pl.ds(base, C)], dstb[q], semi[q]).start()

        def wait_idx(q):
            pltpu.make_async_copy(
                src2_hbm.at[pl.ds(0, C)], srcb[q], semi[q]).wait()
            pltpu.make_async_copy(
                dst_hbm.at[pl.ds(0, C)], dstb[q], semi[q]).wait()

        def g_desc(q, b):
            return pltpu.make_async_copy(h_hbm.at[srcb[q]], stage[b], semg[b])

        def s_desc(q, b):
            return pltpu.make_async_copy(stage[b], acc.at[dstb[q]], sems[b])

        start_idx(0, 0)
        wait_idx(0)
        g_desc(0, 0).start()
        start_idx(1, 1)

        def pair(k2, _):
            for b in (0, 1):
                k = 2 * k2 + b
                g_desc(b, b).wait()
                wait_idx(1 - b)
                g_desc(1 - b, 1 - b).start()
                pltpu.sync_copy(stage[b], acc.at[dstb[b]], add=True)
                start_idx(k + 2, b)
            return 0
        lax.fori_loop(0, NCH // 2, pair, 0)

        g_desc(0, 0).wait()
        wait_idx(1)
        plsc.subcore_barrier()

        pltpu.sync_copy(
            acc.at[pl.ds(sid * ROWS, ROWS)],
            out_hbm.at[pl.ds(cid * N + sid * ROWS, ROWS)],
        )

        @pl.when(sid == NS - 1)
        def _():
            pltpu.sync_copy(
                acc.at[pl.ds(NS * ROWS, N - NS * ROWS)],
                out_hbm.at[pl.ds(cid * N + NS * ROWS, N - NS * ROWS)],
            )

    return sc_agg


_SC_AGG_CACHE = []


def _sc_agg(src2, dst, h):
    if not _SC_AGG_CACHE:
        _SC_AGG_CACHE.append(_make_sc_agg())
    return _SC_AGG_CACHE[0](src2, dst, h)


def _enc_body(xp_ref, w0_ref, b0_ref, w1c_ref, b1c_ref, out_ref):
    t = jnp.maximum(xp_ref[...] @ w0_ref[...] + b0_ref[...], 0.0)
    out_ref[...] = t @ w1c_ref[0] + b1c_ref[0]


def _encoder(xp, w0, b0, w1c, b1c):
    return pl.pallas_call(
        _enc_body,
        grid=(2 * NB,),
        in_specs=[
            pl.BlockSpec((RB, H), lambda j: (j % NB, 0)),
            pl.BlockSpec((H, H), lambda j: (0, 0)),
            pl.BlockSpec((1, H), lambda j: (0, 0)),
            pl.BlockSpec((1, H, HH), lambda j: (j // NB, 0, 0)),
            pl.BlockSpec((1, 1, HH), lambda j: (j // NB, 0, 0)),
        ],
        out_specs=pl.BlockSpec((RB, HH), lambda j: (j, 0)),
        out_shape=jax.ShapeDtypeStruct((2 * N, HH), jnp.float32),
    )(xp, w0, b0, w1c, b1c)


def _mlp_body(a0_ref, a1_ref, h0_ref, h1_ref, wa_ref, ba_ref, wbc_ref, bbc_ref, out_ref):
    z0 = a0_ref[...] + h0_ref[...]
    z1 = a1_ref[...] + h1_ref[...]
    t = jnp.maximum(z0 @ wa_ref[:HH, :] + z1 @ wa_ref[HH:, :] + ba_ref[...], 0.0)
    out_ref[...] = jnp.maximum(t @ wbc_ref[0] + bbc_ref[0], 0.0)


def _gin_mlp(agg, h, wa, ba, wbc, bbc):
    return pl.pallas_call(
        _mlp_body,
        grid=(2 * NB,),
        in_specs=[
            pl.BlockSpec((RB, HH), lambda j: (j % NB, 0)),
            pl.BlockSpec((RB, HH), lambda j: (j % NB + NB, 0)),
            pl.BlockSpec((RB, HH), lambda j: (j % NB, 0)),
            pl.BlockSpec((RB, HH), lambda j: (j % NB + NB, 0)),
            pl.BlockSpec((H, H), lambda j: (0, 0)),
            pl.BlockSpec((1, H), lambda j: (0, 0)),
            pl.BlockSpec((1, H, HH), lambda j: (j // NB, 0, 0)),
            pl.BlockSpec((1, 1, HH), lambda j: (j // NB, 0, 0)),
        ],
        out_specs=pl.BlockSpec((RB, HH), lambda j: (j, 0)),
        out_shape=jax.ShapeDtypeStruct((2 * N, HH), jnp.float32),
    )(agg, agg, h, h, wa, ba, wbc, bbc)


def _readout_body(h0_ref, h1_ref, l1_ref, b1_ref, l2_ref, b2_ref, batch_ref, out_ref):
    j = pl.program_id(0)
    t = jnp.maximum(h0_ref[...] @ l1_ref[:HH, :] + h1_ref[...] @ l1_ref[HH:, :] + b1_ref[...], 0.0)
    y = t @ l2_ref[...] + b2_ref[...]
    b = batch_ref[0, 0, :]
    oh = (b[:, None] == lax.broadcasted_iota(jnp.int32, (RB, G), 1)).astype(jnp.float32)
    contrib = lax.dot_general(oh, y, (((0,), (0,)), ((), ())))

    @pl.when(j == 0)
    def _():
        out_ref[...] = jnp.zeros_like(out_ref)

    out_ref[...] += contrib


def _readout(h, l1, b1, l2, b2, batch3):
    return pl.pallas_call(
        _readout_body,
        grid=(NB,),
        in_specs=[
            pl.BlockSpec((RB, HH), lambda j: (j, 0)),
            pl.BlockSpec((RB, HH), lambda j: (j + NB, 0)),
            pl.BlockSpec((H, HH), lambda j: (0, 0)),
            pl.BlockSpec((1, HH), lambda j: (0, 0)),
            pl.BlockSpec((HH, HH), lambda j: (0, 0)),
            pl.BlockSpec((1, HH), lambda j: (0, 0)),
            pl.BlockSpec((1, 1, RB), lambda j: (j, 0, 0)),
        ],
        out_specs=pl.BlockSpec((G, HH), lambda j: (0, 0)),
        out_shape=jax.ShapeDtypeStruct((G, HH), jnp.float32),
    )(h, h, l1, b1, l2, b2, batch3)


def kernel(x, pos, edge_index, batch, node_W0, node_b0, node_W1, node_b1,
           mlp_Wa, mlp_ba, mlp_Wb, mlp_bb, lin1_W, lin1_b, lin2_W, lin2_b):
    f32 = jnp.float32
    pad = H - (x.shape[1] + pos.shape[1])
    xp = jnp.concatenate([x, pos, jnp.zeros((N, pad), f32)], axis=1)
    w0 = jnp.concatenate([node_W0, jnp.zeros((pad, H), f32)], axis=0)
    w1c = node_W1.reshape(H, 2, HH).transpose(1, 0, 2)
    b0 = node_b0.reshape(1, H)
    b1c = node_b1.reshape(2, 1, HH)

    src = edge_index[0]
    dst = edge_index[1]
    npad = EPAD - E
    if npad:
        spad = jnp.concatenate([src, jnp.zeros((npad,), jnp.int32)])
        dpad = jnp.concatenate([dst, N + (jnp.arange(npad, dtype=jnp.int32) % NGARB)])
    else:
        spad, dpad = src, dst
    src2 = jnp.concatenate([spad, spad + N])
    batch3 = batch.reshape(NB, 1, RB)

    h = _encoder(xp, w0, b0, w1c, b1c)
    for i in range(L):
        agg = _sc_agg(src2, dpad, h)
        wbc = mlp_Wb[i].reshape(H, 2, HH).transpose(1, 0, 2)
        bbc = mlp_bb[i].reshape(2, 1, HH)
        h = _gin_mlp(agg, h, mlp_Wa[i], mlp_ba[i].reshape(1, H), wbc, bbc)

    return _readout(h, lin1_W, lin1_b.reshape(1, HH), lin2_W, lin2_b.reshape(1, HH),
                    batch3)


# speedup vs baseline: 1.4655x; 1.4655x over previous
import functools

import jax
import jax.numpy as jnp
from jax import lax
from jax.experimental import pallas as pl
from jax.experimental.pallas import tpu as pltpu
from jax.experimental.pallas import tpu_sc as plsc

N = 10000
E = 320000
H = 256
HH = 128
G = 64
L = 3

NS = 16
C = 80
NCH = 250
EPC = NCH * C
EPAD = NS * EPC
NGARB = 0
ACCR = N + NGARB
ROWS = 624
ZR = 16
NSTG = 4

RB = 1000
NB = N // RB


def _make_sc_agg():
    mesh = plsc.VectorSubcoreMesh(core_axis_name="c", subcore_axis_name="s")

    @functools.partial(
        pl.kernel,
        mesh=mesh,
        out_type=jax.ShapeDtypeStruct((2 * N, HH), jnp.float32),
        scratch_types=[
            pltpu.VMEM((C,), jnp.int32),
            pltpu.VMEM((C,), jnp.int32),
            pltpu.VMEM((C,), jnp.int32),
            pltpu.VMEM((C,), jnp.int32),
            pltpu.VMEM((C,), jnp.int32),
            pltpu.VMEM((C,), jnp.int32),
            pltpu.VMEM((C,), jnp.int32),
            pltpu.VMEM((C,), jnp.int32),
            pltpu.VMEM((C, HH), jnp.float32),
            pltpu.VMEM((C, HH), jnp.float32),
            pltpu.VMEM((ZR, HH), jnp.float32),
            pltpu.VMEM_SHARED((ACCR, HH), jnp.float32),
            pltpu.SemaphoreType.DMA,
            pltpu.SemaphoreType.DMA,
            pltpu.SemaphoreType.DMA,
            pltpu.SemaphoreType.DMA,
            pltpu.SemaphoreType.DMA,
            pltpu.SemaphoreType.DMA,
            pltpu.SemaphoreType.DMA,
            pltpu.SemaphoreType.DMA,
        ],
    )
    def sc_agg(src2_hbm, dst_hbm, h_hbm, out_hbm,
               srcb0, srcb1, srcb2, srcb3, dstb0, dstb1, dstb2, dstb3,
               stage0, stage1, zbuf, acc,
               semg0, semg1, sems0, sems1, semi0, semi1, semi2, semi3):
        cid = lax.axis_index("c")
        sid = lax.axis_index("s")
        srcb = (srcb0, srcb1, srcb2, srcb3)
        dstb = (dstb0, dstb1, dstb2, dstb3)
        stage = (stage0, stage1)
        semg = (semg0, semg1)
        sems = (sems0, sems1)
        semi = (semi0, semi1, semi2, semi3)

        def zrow(i, _):
            def zcol(j, _):
                zbuf[i, pl.ds(j * 16, 16)] = jnp.zeros((16,), jnp.float32)
                return 0
            return lax.fori_loop(0, HH // 16, zcol, 0)
        lax.fori_loop(0, ZR, zrow, 0)

        def zcp(t, _):
            pltpu.make_async_copy(
                zbuf, acc.at[pl.ds(sid * ROWS + t * ZR, ZR)], sems0).start()
            return 0
        lax.fori_loop(0, ROWS // ZR, zcp, 0)

        @pl.when(sid == NS - 1)
        def _():
            pltpu.make_async_copy(
                zbuf, acc.at[pl.ds(NS * ROWS, N - NS * ROWS)], sems0).start()

        def zwait(t, _):
            pltpu.make_async_copy(
                zbuf, acc.at[pl.ds(0, ZR)], sems0).wait()
            return 0
        lax.fori_loop(0, ROWS // ZR, zwait, 0)

        @pl.when(sid == NS - 1)
        def _():
            pltpu.make_async_copy(
                zbuf, acc.at[pl.ds(0, N - NS * ROWS)], sems0).wait()

        plsc.subcore_barrier()


        def start_idx(k, q):
            kk = jnp.minimum(k, NCH - 1)
            base = sid * EPC + kk * C
            pltpu.make_async_copy(
                src2_hbm.at[pl.ds(cid * EPAD + base, C)], srcb[q], semi[q]).start()
            pltpu.make_async_copy(
                dst_hbm.at[pl.ds(base, C)], dstb[q], semi[q]).start()

        def wait_idx(q):
            pltpu.make_async_copy(
                src2_hbm.at[pl.ds(0, C)], srcb[q], semi[q]).wait()
            pltpu.make_async_copy(
                dst_hbm.at[pl.ds(0, C)], dstb[q], semi[q]).wait()

        def g_desc(q, b):
            return pltpu.make_async_copy(h_hbm.at[srcb[q]], stage[b], semg[b])

        def s_desc(q, b):
            return pltpu.make_async_copy(stage[b], acc.at[dstb[q]], sems[b])

        start_idx(0, 0)
        wait_idx(0)
        g_desc(0, 0).start()
        start_idx(1, 1)

        def pair(k2, _):
            for b in (0, 1):
                k = 2 * k2 + b
                g_desc(b, b).wait()
                wait_idx(1 - b)
                g_desc(1 - b, 1 - b).start()
                pltpu.sync_copy(stage[b], acc.at[dstb[b]], add=True)
                start_idx(k + 2, b)
            return 0
        lax.fori_loop(0, NCH // 2, pair, 0)

        g_desc(0, 0).wait()
        wait_idx(1)
        plsc.subcore_barrier()

        pltpu.sync_copy(
            acc.at[pl.ds(sid * ROWS, ROWS)],
            out_hbm.at[pl.ds(cid * N + sid * ROWS, ROWS)],
        )

        @pl.when(sid == NS - 1)
        def _():
            pltpu.sync_copy(
                acc.at[pl.ds(NS * ROWS, N - NS * ROWS)],
                out_hbm.at[pl.ds(cid * N + NS * ROWS, N - NS * ROWS)],
            )

    return sc_agg


_SC_AGG_CACHE = []


def _sc_agg(src2, dst, h):
    if not _SC_AGG_CACHE:
        _SC_AGG_CACHE.append(_make_sc_agg())
    return _SC_AGG_CACHE[0](src2, dst, h)


def _enc_body(xp_ref, w0_ref, b0_ref, w1c_ref, b1c_ref, out_ref):
    t = jnp.maximum(xp_ref[...] @ w0_ref[...] + b0_ref[...], 0.0)
    out_ref[...] = t @ w1c_ref[0] + b1c_ref[0]


def _encoder(xp, w0, b0, w1c, b1c):
    return pl.pallas_call(
        _enc_body,
        grid=(2 * NB,),
        in_specs=[
            pl.BlockSpec((RB, H), lambda j: (j % NB, 0)),
            pl.BlockSpec((H, H), lambda j: (0, 0)),
            pl.BlockSpec((1, H), lambda j: (0, 0)),
            pl.BlockSpec((1, H, HH), lambda j: (j // NB, 0, 0)),
            pl.BlockSpec((1, 1, HH), lambda j: (j // NB, 0, 0)),
        ],
        out_specs=pl.BlockSpec((RB, HH), lambda j: (j, 0)),
        out_shape=jax.ShapeDtypeStruct((2 * N, HH), jnp.float32),
    )(xp, w0, b0, w1c, b1c)


def _mlp_body(a0_ref, a1_ref, h0_ref, h1_ref, wa_ref, ba_ref, wbc_ref, bbc_ref, out_ref):
    z0 = a0_ref[...] + h0_ref[...]
    z1 = a1_ref[...] + h1_ref[...]
    t = jnp.maximum(z0 @ wa_ref[:HH, :] + z1 @ wa_ref[HH:, :] + ba_ref[...], 0.0)
    out_ref[...] = jnp.maximum(t @ wbc_ref[0] + bbc_ref[0], 0.0)


def _gin_mlp(agg, h, wa, ba, wbc, bbc):
    return pl.pallas_call(
        _mlp_body,
        grid=(2 * NB,),
        in_specs=[
            pl.BlockSpec((RB, HH), lambda j: (j % NB, 0)),
            pl.BlockSpec((RB, HH), lambda j: (j % NB + NB, 0)),
            pl.BlockSpec((RB, HH), lambda j: (j % NB, 0)),
            pl.BlockSpec((RB, HH), lambda j: (j % NB + NB, 0)),
            pl.BlockSpec((H, H), lambda j: (0, 0)),
            pl.BlockSpec((1, H), lambda j: (0, 0)),
            pl.BlockSpec((1, H, HH), lambda j: (j // NB, 0, 0)),
            pl.BlockSpec((1, 1, HH), lambda j: (j // NB, 0, 0)),
        ],
        out_specs=pl.BlockSpec((RB, HH), lambda j: (j, 0)),
        out_shape=jax.ShapeDtypeStruct((2 * N, HH), jnp.float32),
    )(agg, agg, h, h, wa, ba, wbc, bbc)


def _readout_body(a0_ref, a1_ref, h0_ref, h1_ref, wa_ref, ba_ref, wb_ref, bb_ref,
                  l1_ref, b1_ref, l2_ref, b2_ref, batch_ref, out_ref):
    j = pl.program_id(0)
    z0 = a0_ref[...] + h0_ref[...]
    z1 = a1_ref[...] + h1_ref[...]
    t = jnp.maximum(z0 @ wa_ref[:HH, :] + z1 @ wa_ref[HH:, :] + ba_ref[...], 0.0)
    hf = jnp.maximum(t @ wb_ref[...] + bb_ref[...], 0.0)
    r = jnp.maximum(hf[:, :HH] @ l1_ref[:HH, :] + hf[:, HH:] @ l1_ref[HH:, :] + b1_ref[...], 0.0)
    y = r @ l2_ref[...] + b2_ref[...]
    b = batch_ref[0, 0, :]
    oh = (b[:, None] == lax.broadcasted_iota(jnp.int32, (RB, G), 1)).astype(jnp.float32)
    contrib = lax.dot_general(oh, y, (((0,), (0,)), ((), ())))

    @pl.when(j == 0)
    def _():
        out_ref[...] = jnp.zeros_like(out_ref)

    out_ref[...] += contrib


def _readout(agg, h, wa, ba, wb, bb, l1, b1, l2, b2, batch3):
    return pl.pallas_call(
        _readout_body,
        grid=(NB,),
        in_specs=[
            pl.BlockSpec((RB, HH), lambda j: (j, 0)),
            pl.BlockSpec((RB, HH), lambda j: (j + NB, 0)),
            pl.BlockSpec((RB, HH), lambda j: (j, 0)),
            pl.BlockSpec((RB, HH), lambda j: (j + NB, 0)),
            pl.BlockSpec((H, H), lambda j: (0, 0)),
            pl.BlockSpec((1, H), lambda j: (0, 0)),
            pl.BlockSpec((H, H), lambda j: (0, 0)),
            pl.BlockSpec((1, H), lambda j: (0, 0)),
            pl.BlockSpec((H, HH), lambda j: (0, 0)),
            pl.BlockSpec((1, HH), lambda j: (0, 0)),
            pl.BlockSpec((HH, HH), lambda j: (0, 0)),
            pl.BlockSpec((1, HH), lambda j: (0, 0)),
            pl.BlockSpec((1, 1, RB), lambda j: (j, 0, 0)),
        ],
        out_specs=pl.BlockSpec((G, HH), lambda j: (0, 0)),
        out_shape=jax.ShapeDtypeStruct((G, HH), jnp.float32),
    )(agg, agg, h, h, wa, ba, wb, bb, l1, b1, l2, b2, batch3)


def kernel(x, pos, edge_index, batch, node_W0, node_b0, node_W1, node_b1,
           mlp_Wa, mlp_ba, mlp_Wb, mlp_bb, lin1_W, lin1_b, lin2_W, lin2_b):
    f32 = jnp.float32
    pad = H - (x.shape[1] + pos.shape[1])
    xp = jnp.concatenate([x, pos, jnp.zeros((N, pad), f32)], axis=1)
    w0 = jnp.concatenate([node_W0, jnp.zeros((pad, H), f32)], axis=0)
    w1c = node_W1.reshape(H, 2, HH).transpose(1, 0, 2)
    b0 = node_b0.reshape(1, H)
    b1c = node_b1.reshape(2, 1, HH)

    src = edge_index[0]
    dst = edge_index[1]
    npad = EPAD - E
    if npad:
        spad = jnp.concatenate([src, jnp.zeros((npad,), jnp.int32)])
        dpad = jnp.concatenate([dst, N + (jnp.arange(npad, dtype=jnp.int32) % NGARB)])
    else:
        spad, dpad = src, dst
    src2 = jnp.concatenate([spad, spad + N])
    batch3 = batch.reshape(NB, 1, RB)

    h = _encoder(xp, w0, b0, w1c, b1c)
    for i in range(L - 1):
        agg = _sc_agg(src2, dpad, h)
        wbc = mlp_Wb[i].reshape(H, 2, HH).transpose(1, 0, 2)
        bbc = mlp_bb[i].reshape(2, 1, HH)
        h = _gin_mlp(agg, h, mlp_Wa[i], mlp_ba[i].reshape(1, H), wbc, bbc)

    agg = _sc_agg(src2, dpad, h)
    return _readout(agg, h, mlp_Wa[L - 1], mlp_ba[L - 1].reshape(1, H),
                    mlp_Wb[L - 1], mlp_bb[L - 1].reshape(1, H),
                    lin1_W, lin1_b.reshape(1, HH), lin2_W, lin2_b.reshape(1, HH),
                    batch3)
